# baseline (speedup 1.0000x reference)
import functools

import jax
import jax.numpy as jnp
from jax import lax
from jax.experimental import pallas as pl
from jax.experimental.pallas import tpu as pltpu
from jax.experimental.pallas import tpu_sc as plsc

N = 65536
NQ = 1024
K = 16
L = 16
NC = 2
NS = 16
NW = NC * NS
QPW = NQ // NW
CHUNK = 16384
NCHUNKS = N // CHUNK
GROUP = 512
NGROUPS = CHUNK // GROUP
VPG = GROUP // L

_F32 = jnp.float32
_I32 = jnp.int32


def _bf16_round(x):
    u = plsc.bitcast(x, jnp.uint32)
    lsb = lax.shift_right_logical(u, jnp.uint32(16)) & jnp.uint32(1)
    u = (u + jnp.uint32(0x7FFF) + lsb) & jnp.uint32(0xFFFF0000)
    return plsc.bitcast(u, _F32)


def _knn_body(px_hbm, py_hbm, pz_hbm, qx_hbm, qy_hbm, qz_hbm,
              out_idx_hbm, out_dist_hbm,
              px_v, py_v, pz_v, psq_v, pxb_v, pyb_v, pzb_v,
              qx_v, qy_v, qz_v, best_d_v, best_i_v):
    wid = lax.axis_index("s") * NC + lax.axis_index("c")
    qbase = wid * QPW
    iota = lax.broadcasted_iota(_I32, (L,), 0)
    inf_vec = jnp.full((L,), jnp.inf, _F32)

    pltpu.sync_copy(qx_hbm.at[pl.ds(qbase, QPW)], qx_v)
    pltpu.sync_copy(qy_hbm.at[pl.ds(qbase, QPW)], qy_v)
    pltpu.sync_copy(qz_hbm.at[pl.ds(qbase, QPW)], qz_v)

    for j in range(QPW * K // L):
        best_d_v[pl.ds(j * L, L)] = inf_vec
        best_i_v[pl.ds(j * L, L)] = jnp.zeros((L,), _I32)

    def splat(ref, q):
        base = (q // L) * L
        v = ref[pl.ds(base, L)]
        lane = jnp.full((L,), q - base, _I32)
        return v.at[lane].get(mode="promise_in_bounds")

    for ci in range(NCHUNKS):
        pltpu.sync_copy(px_hbm.at[pl.ds(ci * CHUNK, CHUNK)], px_v)
        pltpu.sync_copy(py_hbm.at[pl.ds(ci * CHUNK, CHUNK)], py_v)
        pltpu.sync_copy(pz_hbm.at[pl.ds(ci * CHUNK, CHUNK)], pz_v)

        def prep(i, _):
            ws = []
            for h in range(2):
                x = px_v[pl.ds(i * 2 * L + h * L, L)]
                y = py_v[pl.ds(i * 2 * L + h * L, L)]
                z = pz_v[pl.ds(i * 2 * L + h * L, L)]
                psq_v[pl.ds(i * 2 * L + h * L, L)] = (x * x + z * z) + y * y
                ws.append((2.0 * _bf16_round(x),
                           2.0 * _bf16_round(y),
                           2.0 * _bf16_round(z)))
            fmt = plsc.PackFormat.INTERLEAVED
            pxb_v[pl.ds(i * L, L)] = plsc.bitcast(
                plsc.pack(ws[0][0], ws[1][0], format=fmt), _I32)
            pyb_v[pl.ds(i * L, L)] = plsc.bitcast(
                plsc.pack(ws[0][1], ws[1][1], format=fmt), _I32)
            pzb_v[pl.ds(i * L, L)] = plsc.bitcast(
                plsc.pack(ws[0][2], ws[1][2], format=fmt), _I32)
            return 0

        lax.fori_loop(0, CHUNK // (2 * L), prep, 0)

        def qbody(q, _, ci=ci):
            qxs = splat(qx_v, q)
            qys = splat(qy_v, q)
            qzs = splat(qz_v, q)
            qsq = (qxs * qxs + qzs * qzs) + qys * qys
            qxb = _bf16_round(qxs)
            qyb = _bf16_round(qys)
            qzb = _bf16_round(qzs)
            tvec0 = jnp.full((L,), jnp.max(best_d_v[pl.ds(q * K, K)]), _F32)

            def gbody(g, tvec):
                off = g * GROUP
                off2 = off // 2
                fmt = plsc.PackFormat.INTERLEAVED
                d = []
                for jp in range(VPG // 2):
                    xs = plsc.unpack(plsc.bitcast(
                        pxb_v[pl.ds(off2 + jp * L, L)], jnp.bfloat16), format=fmt)
                    ys = plsc.unpack(plsc.bitcast(
                        pyb_v[pl.ds(off2 + jp * L, L)], jnp.bfloat16), format=fmt)
                    zs = plsc.unpack(plsc.bitcast(
                        pzb_v[pl.ds(off2 + jp * L, L)], jnp.bfloat16), format=fmt)
                    for h in range(2):
                        j = jp * 2 + h
                        dot2 = xs[h] * qxb + ys[h] * qyb
                        dot2 = dot2 + zs[h] * qzb
                        a = qsq - dot2
                        d.append(a + psq_v[pl.ds(off + j * L, L)])
                mins = list(d)
                while len(mins) > 1:
                    mins = [jnp.minimum(mins[2 * t], mins[2 * t + 1])
                            for t in range(len(mins) // 2)]
                gmin = mins[0]
                hit = jnp.any(gmin < tvec)

                def merge(tvec):
                    del tvec
                    base = ci * CHUNK + g * GROUP
                    def bmerge(a, b):
                        rd = lax.rev(b[0], (0,))
                        ri = lax.rev(b[1], (0,))
                        m = rd < a[0]
                        md = jnp.where(m, rd, a[0])
                        mi = jnp.where(m, ri, a[1])
                        nd, ni = plsc.sort_key_val(md, mi)
                        return (nd, ni)

                    runs = []
                    for j in range(VPG):
                        ij = jnp.full((L,), base + j * L, _I32) + iota
                        sd, si = plsc.sort_key_val(d[j], ij)
                        runs.append((sd, si))
                    while len(runs) > 1:
                        runs = [bmerge(runs[2 * t], runs[2 * t + 1])
                                for t in range(len(runs) // 2)]
                    bd = best_d_v[pl.ds(q * K, K)]
                    bi = best_i_v[pl.ds(q * K, K)]
                    bd, bi = bmerge((bd, bi), runs[0])
                    best_d_v[pl.ds(q * K, K)] = bd
                    best_i_v[pl.ds(q * K, K)] = bi
                    return jnp.full((L,), jnp.max(bd), _F32)

                return lax.cond(hit, merge, lambda t: t, tvec)

            lax.fori_loop(0, NGROUPS, gbody, tvec0)
            return 0

        lax.fori_loop(0, QPW, qbody, 0)

    pltpu.sync_copy(best_i_v, out_idx_hbm.at[pl.ds(qbase * K, QPW * K)])
    pltpu.sync_copy(best_d_v, out_dist_hbm.at[pl.ds(qbase * K, QPW * K)])


@jax.jit
def _knn(px, py, pz, qx, qy, qz):
    mesh = plsc.VectorSubcoreMesh(core_axis_name="c", subcore_axis_name="s")
    f = pl.kernel(
        _knn_body,
        out_type=[
            jax.ShapeDtypeStruct((NQ * K,), _I32),
            jax.ShapeDtypeStruct((NQ * K,), _F32),
        ],
        mesh=mesh,
        compiler_params=pltpu.CompilerParams(needs_layout_passes=False),
        scratch_types=[
            pltpu.VMEM((CHUNK,), _F32),
            pltpu.VMEM((CHUNK,), _F32),
            pltpu.VMEM((CHUNK,), _F32),
            pltpu.VMEM((CHUNK,), _F32),
            pltpu.VMEM((CHUNK // 2,), _I32),
            pltpu.VMEM((CHUNK // 2,), _I32),
            pltpu.VMEM((CHUNK // 2,), _I32),
            pltpu.VMEM((QPW,), _F32),
            pltpu.VMEM((QPW,), _F32),
            pltpu.VMEM((QPW,), _F32),
            pltpu.VMEM((QPW * K,), _F32),
            pltpu.VMEM((QPW * K,), _I32),
        ],
    )
    return f(px, py, pz, qx, qy, qz)


def kernel(points, queries, k):
    del k
    assert points.shape == (N, 3) and queries.shape == (NQ, 3)
    px, py, pz = points[:, 0], points[:, 1], points[:, 2]
    qx, qy, qz = queries[:, 0], queries[:, 1], queries[:, 2]
    idx, dist = _knn(px, py, pz, qx, qy, qz)
    row_splits = jnp.arange(NQ + 1, dtype=_I32) * K
    return idx, row_splits, dist


# speedup vs baseline: 1.0163x; 1.0163x over previous
import jax
import jax.numpy as jnp
from jax import lax
from jax.experimental import pallas as pl
from jax.experimental.pallas import tpu as pltpu
from jax.experimental.pallas import tpu_sc as plsc

N = 65536
NQ = 1024
K = 16
L = 16
NC = 2
NS = 16
NW = NC * NS
QPW = NQ // NW
CHUNK = 16384
NCHUNKS = N // CHUNK
GROUP = 512
NGROUPS = CHUNK // GROUP
VPG = GROUP // L

_F32 = jnp.float32
_I32 = jnp.int32


def _bf16_round(x):
    u = plsc.bitcast(x, jnp.uint32)
    lsb = lax.shift_right_logical(u, jnp.uint32(16)) & jnp.uint32(1)
    u = (u + jnp.uint32(0x7FFF) + lsb) & jnp.uint32(0xFFFF0000)
    return plsc.bitcast(u, _F32)


def _knn_body(px_hbm, py_hbm, pz_hbm, qx_hbm, qy_hbm, qz_hbm,
              out_idx_hbm, out_dist_hbm,
              px_v, py_v, pz_v, psq_v, pxb_v, pyb_v, pzb_v,
              qx_v, qy_v, qz_v, best_d_v, best_i_v):
    wid = lax.axis_index("s") * NC + lax.axis_index("c")
    qbase = wid * QPW
    iota = lax.broadcasted_iota(_I32, (L,), 0)
    inf_vec = jnp.full((L,), jnp.inf, _F32)

    pltpu.sync_copy(qx_hbm.at[pl.ds(qbase, QPW)], qx_v)
    pltpu.sync_copy(qy_hbm.at[pl.ds(qbase, QPW)], qy_v)
    pltpu.sync_copy(qz_hbm.at[pl.ds(qbase, QPW)], qz_v)

    for j in range(QPW * K // L):
        best_d_v[pl.ds(j * L, L)] = inf_vec
        best_i_v[pl.ds(j * L, L)] = jnp.zeros((L,), _I32)

    def splat(ref, q):
        base = (q // L) * L
        v = ref[pl.ds(base, L)]
        lane = jnp.full((L,), q - base, _I32)
        return v.at[lane].get(mode="promise_in_bounds")

    for ci in range(NCHUNKS):
        pltpu.sync_copy(px_hbm.at[pl.ds(ci * CHUNK, CHUNK)], px_v)
        pltpu.sync_copy(py_hbm.at[pl.ds(ci * CHUNK, CHUNK)], py_v)
        pltpu.sync_copy(pz_hbm.at[pl.ds(ci * CHUNK, CHUNK)], pz_v)

        def prep(i, _):
            ws = []
            for h in range(2):
                x = px_v[pl.ds(i * 2 * L + h * L, L)]
                y = py_v[pl.ds(i * 2 * L + h * L, L)]
                z = pz_v[pl.ds(i * 2 * L + h * L, L)]
                psq_v[pl.ds(i * 2 * L + h * L, L)] = (x * x + z * z) + y * y
                ws.append((2.0 * _bf16_round(x),
                           2.0 * _bf16_round(y),
                           2.0 * _bf16_round(z)))
            fmt = plsc.PackFormat.INTERLEAVED
            pxb_v[pl.ds(i * L, L)] = plsc.bitcast(
                plsc.pack(ws[0][0], ws[1][0], format=fmt), _I32)
            pyb_v[pl.ds(i * L, L)] = plsc.bitcast(
                plsc.pack(ws[0][1], ws[1][1], format=fmt), _I32)
            pzb_v[pl.ds(i * L, L)] = plsc.bitcast(
                plsc.pack(ws[0][2], ws[1][2], format=fmt), _I32)
            return 0

        lax.fori_loop(0, CHUNK // (2 * L), prep, 0)

        def qbody(q, _, ci=ci):
            qxs = splat(qx_v, q)
            qys = splat(qy_v, q)
            qzs = splat(qz_v, q)
            qsq = (qxs * qxs + qzs * qzs) + qys * qys
            qxb = _bf16_round(qxs)
            qyb = _bf16_round(qys)
            qzb = _bf16_round(qzs)
            tvec0 = jnp.full((L,), jnp.max(best_d_v[pl.ds(q * K, K)]), _F32)

            def gbody(g, tvec):
                off = g * GROUP
                off2 = off // 2
                fmt = plsc.PackFormat.INTERLEAVED
                d = []
                for jp in range(VPG // 2):
                    xs = plsc.unpack(plsc.bitcast(
                        pxb_v[pl.ds(off2 + jp * L, L)], jnp.bfloat16), format=fmt)
                    ys = plsc.unpack(plsc.bitcast(
                        pyb_v[pl.ds(off2 + jp * L, L)], jnp.bfloat16), format=fmt)
                    zs = plsc.unpack(plsc.bitcast(
                        pzb_v[pl.ds(off2 + jp * L, L)], jnp.bfloat16), format=fmt)
                    for h in range(2):
                        j = jp * 2 + h
                        dot2 = xs[h] * qxb + ys[h] * qyb
                        dot2 = dot2 + zs[h] * qzb
                        a = qsq - dot2
                        d.append(a + psq_v[pl.ds(off + j * L, L)])
                gmin = d[0]
                for j in range(1, VPG):
                    gmin = jnp.minimum(gmin, d[j])
                hit = jnp.any(gmin < tvec)

                def merge(tvec):
                    del tvec
                    base = ci * CHUNK + g * GROUP
                    def bmerge(a, b):
                        rd = lax.rev(b[0], (0,))
                        ri = lax.rev(b[1], (0,))
                        m = rd < a[0]
                        md = jnp.where(m, rd, a[0])
                        mi = jnp.where(m, ri, a[1])
                        nd, ni = plsc.sort_key_val(md, mi)
                        return (nd, ni)

                    runs = []
                    for j in range(VPG):
                        ij = jnp.full((L,), base + j * L, _I32) + iota
                        sd, si = plsc.sort_key_val(d[j], ij)
                        runs.append((sd, si))
                    while len(runs) > 1:
                        runs = [bmerge(runs[2 * t], runs[2 * t + 1])
                                for t in range(len(runs) // 2)]
                    bd = best_d_v[pl.ds(q * K, K)]
                    bi = best_i_v[pl.ds(q * K, K)]
                    bd, bi = bmerge((bd, bi), runs[0])
                    best_d_v[pl.ds(q * K, K)] = bd
                    best_i_v[pl.ds(q * K, K)] = bi
                    return jnp.full((L,), jnp.max(bd), _F32)

                return lax.cond(hit, merge, lambda t: t, tvec)

            lax.fori_loop(0, NGROUPS, gbody, tvec0)
            return 0

        lax.fori_loop(0, QPW, qbody, 0)

    pltpu.sync_copy(best_i_v, out_idx_hbm.at[pl.ds(qbase * K, QPW * K)])
    pltpu.sync_copy(best_d_v, out_dist_hbm.at[pl.ds(qbase * K, QPW * K)])


@jax.jit
def _knn(px, py, pz, qx, qy, qz):
    mesh = plsc.VectorSubcoreMesh(core_axis_name="c", subcore_axis_name="s")
    f = pl.kernel(
        _knn_body,
        out_type=[
            jax.ShapeDtypeStruct((NQ * K,), _I32),
            jax.ShapeDtypeStruct((NQ * K,), _F32),
        ],
        mesh=mesh,
        compiler_params=pltpu.CompilerParams(needs_layout_passes=False),
        scratch_types=[
            pltpu.VMEM((CHUNK,), _F32),
            pltpu.VMEM((CHUNK,), _F32),
            pltpu.VMEM((CHUNK,), _F32),
            pltpu.VMEM((CHUNK,), _F32),
            pltpu.VMEM((CHUNK // 2,), _I32),
            pltpu.VMEM((CHUNK // 2,), _I32),
            pltpu.VMEM((CHUNK // 2,), _I32),
            pltpu.VMEM((QPW,), _F32),
            pltpu.VMEM((QPW,), _F32),
            pltpu.VMEM((QPW,), _F32),
            pltpu.VMEM((QPW * K,), _F32),
            pltpu.VMEM((QPW * K,), _I32),
        ],
    )
    return f(px, py, pz, qx, qy, qz)


def kernel(points, queries, k):
    del k
    assert points.shape == (N, 3) and queries.shape == (NQ, 3)
    px, py, pz = points[:, 0], points[:, 1], points[:, 2]
    qx, qy, qz = queries[:, 0], queries[:, 1], queries[:, 2]
    idx, dist = _knn(px, py, pz, qx, qy, qz)
    row_splits = jnp.arange(NQ + 1, dtype=_I32) * K
    return idx, row_splits, dist


# speedup vs baseline: 1.0264x; 1.0099x over previous
import jax
import jax.numpy as jnp
from jax import lax
from jax.experimental import pallas as pl
from jax.experimental.pallas import tpu as pltpu
from jax.experimental.pallas import tpu_sc as plsc

N = 65536
NQ = 1024
K = 16
L = 16
NC = 2
NS = 16
NW = NC * NS
QPW = NQ // NW
CHUNK = 16384
NCHUNKS = N // CHUNK
GROUP = 512
NGROUPS = CHUNK // GROUP
VPG = GROUP // L

_F32 = jnp.float32
_I32 = jnp.int32


def _bf16_round(x):
    u = plsc.bitcast(x, jnp.uint32)
    lsb = lax.shift_right_logical(u, jnp.uint32(16)) & jnp.uint32(1)
    u = (u + jnp.uint32(0x7FFF) + lsb) & jnp.uint32(0xFFFF0000)
    return plsc.bitcast(u, _F32)


def _knn_body(px_hbm, py_hbm, pz_hbm, qx_hbm, qy_hbm, qz_hbm,
              out_idx_hbm, out_dist_hbm,
              px_v, py_v, pz_v, psq_v, pxb_v, pyb_v, pzb_v,
              qx_v, qy_v, qz_v, best_d_v, best_i_v):
    wid = lax.axis_index("s") * NC + lax.axis_index("c")
    qbase = wid * QPW
    iota = lax.broadcasted_iota(_I32, (L,), 0)
    inf_vec = jnp.full((L,), jnp.inf, _F32)

    pltpu.sync_copy(qx_hbm.at[pl.ds(qbase, QPW)], qx_v)
    pltpu.sync_copy(qy_hbm.at[pl.ds(qbase, QPW)], qy_v)
    pltpu.sync_copy(qz_hbm.at[pl.ds(qbase, QPW)], qz_v)

    for j in range(QPW * K // L):
        best_d_v[pl.ds(j * L, L)] = inf_vec
        best_i_v[pl.ds(j * L, L)] = jnp.zeros((L,), _I32)

    def splat(ref, q):
        base = (q // L) * L
        v = ref[pl.ds(base, L)]
        lane = jnp.full((L,), q - base, _I32)
        return v.at[lane].get(mode="promise_in_bounds")

    def chunk_body(ci, _):
        pltpu.sync_copy(px_hbm.at[pl.ds(ci * CHUNK, CHUNK)], px_v)
        pltpu.sync_copy(py_hbm.at[pl.ds(ci * CHUNK, CHUNK)], py_v)
        pltpu.sync_copy(pz_hbm.at[pl.ds(ci * CHUNK, CHUNK)], pz_v)

        def prep(i, _):
            ws = []
            for h in range(2):
                x = px_v[pl.ds(i * 2 * L + h * L, L)]
                y = py_v[pl.ds(i * 2 * L + h * L, L)]
                z = pz_v[pl.ds(i * 2 * L + h * L, L)]
                psq_v[pl.ds(i * 2 * L + h * L, L)] = (x * x + z * z) + y * y
                ws.append((2.0 * _bf16_round(x),
                           2.0 * _bf16_round(y),
                           2.0 * _bf16_round(z)))
            fmt = plsc.PackFormat.INTERLEAVED
            pxb_v[pl.ds(i * L, L)] = plsc.bitcast(
                plsc.pack(ws[0][0], ws[1][0], format=fmt), _I32)
            pyb_v[pl.ds(i * L, L)] = plsc.bitcast(
                plsc.pack(ws[0][1], ws[1][1], format=fmt), _I32)
            pzb_v[pl.ds(i * L, L)] = plsc.bitcast(
                plsc.pack(ws[0][2], ws[1][2], format=fmt), _I32)
            return 0

        lax.fori_loop(0, CHUNK // (2 * L), prep, 0)

        def qbody(q, _):
            qxs = splat(qx_v, q)
            qys = splat(qy_v, q)
            qzs = splat(qz_v, q)
            qsq = (qxs * qxs + qzs * qzs) + qys * qys
            qxb = _bf16_round(qxs)
            qyb = _bf16_round(qys)
            qzb = _bf16_round(qzs)
            tvec0 = jnp.full((L,), jnp.max(best_d_v[pl.ds(q * K, K)]), _F32)

            def do_group(g, tvec):
                off = g * GROUP
                off2 = off // 2
                fmt = plsc.PackFormat.INTERLEAVED
                d = []
                for jp in range(VPG // 2):
                    xs = plsc.unpack(plsc.bitcast(
                        pxb_v[pl.ds(off2 + jp * L, L)], jnp.bfloat16), format=fmt)
                    ys = plsc.unpack(plsc.bitcast(
                        pyb_v[pl.ds(off2 + jp * L, L)], jnp.bfloat16), format=fmt)
                    zs = plsc.unpack(plsc.bitcast(
                        pzb_v[pl.ds(off2 + jp * L, L)], jnp.bfloat16), format=fmt)
                    for h in range(2):
                        j = jp * 2 + h
                        dot2 = xs[h] * qxb + ys[h] * qyb
                        dot2 = dot2 + zs[h] * qzb
                        a = qsq - dot2
                        d.append(a + psq_v[pl.ds(off + j * L, L)])
                gmin = d[0]
                for j in range(1, VPG):
                    gmin = jnp.minimum(gmin, d[j])
                hit = jnp.any(gmin < tvec)

                def merge(tvec):
                    del tvec
                    base = ci * CHUNK + g * GROUP
                    def bmerge(a, b):
                        rd = lax.rev(b[0], (0,))
                        ri = lax.rev(b[1], (0,))
                        m = rd < a[0]
                        md = jnp.where(m, rd, a[0])
                        mi = jnp.where(m, ri, a[1])
                        nd, ni = plsc.sort_key_val(md, mi)
                        return (nd, ni)

                    runs = []
                    for j in range(VPG):
                        ij = jnp.full((L,), base + j * L, _I32) + iota
                        sd, si = plsc.sort_key_val(d[j], ij)
                        runs.append((sd, si))
                    while len(runs) > 1:
                        runs = [bmerge(runs[2 * t], runs[2 * t + 1])
                                for t in range(len(runs) // 2)]
                    bd = best_d_v[pl.ds(q * K, K)]
                    bi = best_i_v[pl.ds(q * K, K)]
                    bd, bi = bmerge((bd, bi), runs[0])
                    best_d_v[pl.ds(q * K, K)] = bd
                    best_i_v[pl.ds(q * K, K)] = bi
                    return jnp.full((L,), jnp.max(bd), _F32)

                return lax.cond(hit, merge, lambda t: t, tvec)

            def gbody(g2, tvec):
                tvec = do_group(2 * g2, tvec)
                return do_group(2 * g2 + 1, tvec)

            lax.fori_loop(0, NGROUPS // 2, gbody, tvec0)
            return 0

        lax.fori_loop(0, QPW, qbody, 0)
        return 0

    lax.fori_loop(0, NCHUNKS, chunk_body, 0)

    pltpu.sync_copy(best_i_v, out_idx_hbm.at[pl.ds(qbase * K, QPW * K)])
    pltpu.sync_copy(best_d_v, out_dist_hbm.at[pl.ds(qbase * K, QPW * K)])


@jax.jit
def _knn(px, py, pz, qx, qy, qz):
    mesh = plsc.VectorSubcoreMesh(core_axis_name="c", subcore_axis_name="s")
    f = pl.kernel(
        _knn_body,
        out_type=[
            jax.ShapeDtypeStruct((NQ * K,), _I32),
            jax.ShapeDtypeStruct((NQ * K,), _F32),
        ],
        mesh=mesh,
        compiler_params=pltpu.CompilerParams(needs_layout_passes=False),
        scratch_types=[
            pltpu.VMEM((CHUNK,), _F32),
            pltpu.VMEM((CHUNK,), _F32),
            pltpu.VMEM((CHUNK,), _F32),
            pltpu.VMEM((CHUNK,), _F32),
            pltpu.VMEM((CHUNK // 2,), _I32),
            pltpu.VMEM((CHUNK // 2,), _I32),
            pltpu.VMEM((CHUNK // 2,), _I32),
            pltpu.VMEM((QPW,), _F32),
            pltpu.VMEM((QPW,), _F32),
            pltpu.VMEM((QPW,), _F32),
            pltpu.VMEM((QPW * K,), _F32),
            pltpu.VMEM((QPW * K,), _I32),
        ],
    )
    return f(px, py, pz, qx, qy, qz)


def kernel(points, queries, k):
    del k
    assert points.shape == (N, 3) and queries.shape == (NQ, 3)
    px, py, pz = points[:, 0], points[:, 1], points[:, 2]
    qx, qy, qz = queries[:, 0], queries[:, 1], queries[:, 2]
    idx, dist = _knn(px, py, pz, qx, qy, qz)
    row_splits = jnp.arange(NQ + 1, dtype=_I32) * K
    return idx, row_splits, dist
